# baseline (speedup 1.0000x reference)
import functools

import jax
import jax.numpy as jnp
import numpy as np
from jax import lax
from jax.experimental import pallas as pl
from jax.experimental.pallas import tpu as pltpu
from jax.experimental.pallas import tpu_sc as plsc

_K = 16
_EPS = 1e-6
_HI = lax.Precision.HIGHEST
_NC, _NS = 2, 16
_NW = _NC * _NS


def _sc_gather(table, idx):
    m0 = idx.shape[0]
    d = table.shape[1]
    mpad = -(-m0 // (16 * _NW)) * (16 * _NW)
    if mpad != m0:
        idx = jnp.concatenate([idx, jnp.zeros((mpad - m0,), jnp.int32)])
    b_per_w = mpad // _NW
    ch = b_per_w
    while ch * (d + 1) > 65536:
        ch //= 2
    nchunk = b_per_w // ch
    mesh = plsc.VectorSubcoreMesh(core_axis_name="c", subcore_axis_name="s",
                                  num_cores=_NC, num_subcores=_NS)

    @functools.partial(
        pl.kernel,
        out_type=jax.ShapeDtypeStruct((mpad, d), jnp.float32),
        mesh=mesh,
        compiler_params=pltpu.CompilerParams(use_tc_tiling_on_sc=False),
        scratch_types=[
            pltpu.VMEM((ch,), jnp.int32),
            pltpu.VMEM((ch, d), jnp.float32),
            pltpu.SemaphoreType.DMA,
        ],
    )
    def gath(table_hbm, idx_hbm, out_hbm, idx_v, rows_v, sem):
        wid = lax.axis_index("s") * _NC + lax.axis_index("c")
        base = wid * b_per_w
        for c in range(nchunk):
            pltpu.sync_copy(idx_hbm.at[pl.ds(base + c * ch, ch)], idx_v)
            pltpu.async_copy(table_hbm.at[idx_v], rows_v, sem).wait()
            pltpu.sync_copy(rows_v, out_hbm.at[pl.ds(base + c * ch, ch)])

    out = gath(table, idx)
    return out[:m0] if mpad != m0 else out


def _linear(xs, ws, bias, act):
    m = xs[0].shape[0]
    co = ws[0].shape[1]
    tm = min(m, 512)
    nin = len(xs)
    b2 = bias.reshape(1, co)

    def body(*refs):
        o_ref = refs[-1]
        tot = None
        for i in range(nin):
            t = jnp.dot(refs[i][...], refs[nin + i][...],
                        preferred_element_type=jnp.float32, precision=_HI)
            tot = t if tot is None else tot + t
        tot = tot + refs[2 * nin][...]
        if act == "relu":
            tot = jnp.maximum(tot, 0.0)
        elif act == "lrelu2":
            tot = jnp.where(tot >= 0, tot, 0.2 * tot)
        elif act == "lrelu01":
            tot = jnp.where(tot >= 0, tot, 0.01 * tot)
        o_ref[...] = tot

    in_specs = ([pl.BlockSpec((tm, x.shape[1]), lambda i: (i, 0)) for x in xs]
                + [pl.BlockSpec(w.shape, lambda i: (0, 0)) for w in ws]
                + [pl.BlockSpec((1, co), lambda i: (0, 0))])
    return pl.pallas_call(
        body, grid=(m // tm,),
        in_specs=in_specs,
        out_specs=pl.BlockSpec((tm, co), lambda i: (i, 0)),
        out_shape=jax.ShapeDtypeStruct((m, co), jnp.float32),
    )(*xs, *ws, b2)


def _knn(qa, sa_t, k):
    b, nq, _ = qa.shape
    ns = sa_t.shape[2]
    tq = min(nq, 512)
    grid = (b, nq // tq)
    big = float("inf")

    if k == 1:
        def body1(qa_ref, sa_ref, idx_ref):
            d2 = jnp.dot(qa_ref[0], sa_ref[0],
                         preferred_element_type=jnp.float32, precision=_HI)
            iota = lax.broadcasted_iota(jnp.int32, (tq, ns), 1)
            m = jnp.min(d2, axis=1, keepdims=True)
            idx_ref[0] = jnp.min(jnp.where(d2 == m, iota, ns), axis=1,
                                 keepdims=True)

        return pl.pallas_call(
            body1, grid=grid,
            in_specs=[pl.BlockSpec((1, tq, 5), lambda bb, i: (bb, i, 0)),
                      pl.BlockSpec((1, 5, ns), lambda bb, i: (bb, 0, 0))],
            out_specs=pl.BlockSpec((1, tq, 1), lambda bb, i: (bb, i, 0)),
            out_shape=jax.ShapeDtypeStruct((b, nq, 1), jnp.int32),
        )(qa, sa_t)

    def body(qa_ref, sa_ref, idx_ref, dist_ref):
        d2 = jnp.dot(qa_ref[0], sa_ref[0],
                     preferred_element_type=jnp.float32, precision=_HI)
        ns4 = ns // 4
        iota = lax.broadcasted_iota(jnp.int32, (tq, ns4), 1)
        vs = [d2[:, c * ns4:(c + 1) * ns4] for c in range(4)]
        ids = [iota + c * ns4 for c in range(4)]

        def ce(i, j):
            sel = vs[i] <= vs[j]
            vs[i], vs[j] = (jnp.where(sel, vs[i], vs[j]),
                            jnp.where(sel, vs[j], vs[i]))
            ids[i], ids[j] = (jnp.where(sel, ids[i], ids[j]),
                              jnp.where(sel, ids[j], ids[i]))

        for i, j in ((0, 1), (2, 3), (0, 2), (1, 3), (1, 2)):
            ce(i, j)
        s0, s1, s2, s3 = vs
        i0, i1, i2 = ids[0], ids[1], ids[2]
        for t in range(k):
            m = jnp.min(s0, axis=1, keepdims=True)
            eq = s0 == m
            ii = jnp.min(jnp.where(eq, i0, ns), axis=1, keepdims=True)
            idx_ref[0, :, t:t + 1] = ii
            dist_ref[0, :, t:t + 1] = jnp.where(
                m > 0, jnp.sqrt(jnp.maximum(m, 0.0)), 0.0)
            if t < k - 1:
                s0 = jnp.where(eq, s1, s0)
                i0 = jnp.where(eq, i1, i0)
                s1 = jnp.where(eq, s2, s1)
                i1 = jnp.where(eq, i2, i1)
                s2 = jnp.where(eq, s3, s2)
                i2 = jnp.where(eq, ids[3], i2)
                s3 = jnp.where(eq, big, s3)

    return pl.pallas_call(
        body, grid=grid,
        in_specs=[pl.BlockSpec((1, tq, 5), lambda bb, i: (bb, i, 0)),
                  pl.BlockSpec((1, 5, ns), lambda bb, i: (bb, 0, 0))],
        out_specs=[pl.BlockSpec((1, tq, k), lambda bb, i: (bb, i, 0)),
                   pl.BlockSpec((1, tq, k), lambda bb, i: (bb, i, 0))],
        out_shape=[jax.ShapeDtypeStruct((b, nq, k), jnp.int32),
                   jax.ShapeDtypeStruct((b, nq, k), jnp.float32)],
    )(qa, sa_t)


def _lfa_fused(ext, nbr, dist, f_in, w1t, b1, args1, args2, w2t, wst, btot):
    m, d_in = f_in.shape
    co = w2t.shape[1]
    tp = min(m, 256)
    flat = [w1t, b1, *args1, *args2, w2t, wst, btot]

    def body(*refs):
        ext_ref, nbr_ref, dist_ref, f_ref = refs[:4]
        wrefs = refs[4:-1]
        o_ref = refs[-1]
        w1t_r, b1_r = wrefs[0:2]
        pa1 = wrefs[2:13]
        pa2 = wrefs[13:24]
        w2t_r, wst_r, btot_r = wrefs[24:27]
        ex = ext_ref[...]
        f = f_ref[...]
        x1 = jnp.dot(f, w1t_r[...], preferred_element_type=jnp.float32,
                     precision=_HI) + b1_r[...]
        x1 = jnp.where(x1 >= 0, x1, 0.2 * x1)

        def pool(ft, pa):
            (a_r, bm_r, wd_r, esc_r, esh_r, w1_r, w2_r, wml_r, wmr_r,
             psc_r, psh_r) = pa
            ext_a = jnp.dot(ex, a_r[...], preferred_element_type=jnp.float32,
                            precision=_HI)
            f_w2 = jnp.dot(ft, w2_r[...], preferred_element_type=jnp.float32,
                           precision=_HI)
            encs, logs = [], []
            for kk in range(_K):
                nb = nbr_ref[kk, :, 0:3]
                dk = dist_ref[:, kk:kk + 1]
                enc = ext_a + jnp.dot(nb, bm_r[...],
                                      preferred_element_type=jnp.float32,
                                      precision=_HI) + dk * wd_r[...]
                enc = jnp.maximum(enc * esc_r[...] + esh_r[...], 0.0)
                encs.append(enc)
                logs.append(jnp.dot(enc, w1_r[...],
                                    preferred_element_type=jnp.float32,
                                    precision=_HI) + f_w2)
            mx = logs[0]
            for kk in range(1, _K):
                mx = jnp.maximum(mx, logs[kk])
            den = None
            accl = None
            for kk in range(_K):
                e = jnp.exp(logs[kk] - mx)
                den = e if den is None else den + e
                t = e * encs[kk]
                accl = t if accl is None else accl + t
            out = (jnp.dot(accl / den, wml_r[...],
                           preferred_element_type=jnp.float32, precision=_HI)
                   + jnp.dot(ft, wmr_r[...],
                             preferred_element_type=jnp.float32,
                             precision=_HI))
            return jnp.maximum(out * psc_r[...] + psh_r[...], 0.0)

        z1 = pool(x1, pa1)
        z2 = pool(z1, pa2)
        out = (jnp.dot(z2, w2t_r[...], preferred_element_type=jnp.float32,
                       precision=_HI)
               + jnp.dot(f, wst_r[...], preferred_element_type=jnp.float32,
                         precision=_HI) + btot_r[...])
        o_ref[...] = jnp.where(out >= 0, out, 0.01 * out)

    full = lambda arr: pl.BlockSpec(arr.shape, lambda i: tuple(0 for _ in arr.shape))
    in_specs = [pl.BlockSpec((tp, 3), lambda i: (i, 0)),
                pl.BlockSpec((_K, tp, 16), lambda i: (0, i, 0)),
                pl.BlockSpec((tp, _K), lambda i: (i, 0)),
                pl.BlockSpec((tp, d_in), lambda i: (i, 0))] + [full(w) for w in flat]
    return pl.pallas_call(
        body, grid=(m // tp,),
        in_specs=in_specs,
        out_specs=pl.BlockSpec((tp, co), lambda i: (i, 0)),
        out_shape=jax.ShapeDtypeStruct((m, co), jnp.float32),
    )(ext, nbr, dist, f_in, *flat)


def _pool_args(lse_mlp, pool, h):
    w = lse_mlp["W"]
    a = (w[:, 0:3] + w[:, 6:9]).T
    bm = (w[:, 3:6] - w[:, 6:9]).T
    wd = w[:, 9][None, :]
    esc = (lse_mlp["gamma"] / jnp.sqrt(1.0 + _EPS))[None, :]
    esh = (lse_mlp["b"] * esc[0] + lse_mlp["beta"])[None, :]
    wst = pool["score_W"].T
    w1 = wst[:h, :h]
    w2 = wst[h:, :h]
    pm = pool["mlp"]
    wmt = pm["W"].T
    psc = (pm["gamma"] / jnp.sqrt(1.0 + _EPS))[None, :]
    psh = (pm["b"] * psc[0] + pm["beta"])[None, :]
    return (a, bm, wd, esc, esh, w1, w2, wmt[:h], wmt[h:], psc, psh)


def _fc_end(x, w1, b1, w2, b2, w3, b3):
    m = x.shape[0]
    tm = min(m, 1024)

    def body(x_ref, w1_ref, b1_ref, w2_ref, b2_ref, w3_ref, b3_ref, o_ref):
        t = jnp.maximum(jnp.dot(x_ref[...], w1_ref[...],
                                preferred_element_type=jnp.float32,
                                precision=_HI) + b1_ref[...], 0.0)
        t = jnp.maximum(jnp.dot(t, w2_ref[...],
                                preferred_element_type=jnp.float32,
                                precision=_HI) + b2_ref[...], 0.0)
        o_ref[...] = jnp.dot(t, w3_ref[...],
                             preferred_element_type=jnp.float32,
                             precision=_HI) + b3_ref[...]

    full = lambda arr: pl.BlockSpec(arr.shape, lambda i: tuple(0 for _ in arr.shape))
    return pl.pallas_call(
        body, grid=(m // tm,),
        in_specs=[pl.BlockSpec((tm, x.shape[1]), lambda i: (i, 0)),
                  full(w1), full(b1), full(w2), full(b2), full(w3), full(b3)],
        out_specs=pl.BlockSpec((tm, 13), lambda i: (i, 0)),
        out_shape=jax.ShapeDtypeStruct((m, 13), jnp.float32),
    )(x, w1, b1, w2, b2, w3, b3)


@functools.lru_cache(maxsize=None)
def _perm_const(n):
    with jax.ensure_compile_time_eval():
        p = np.asarray(jax.random.permutation(jax.random.key(1), n))
    return p.astype(np.int32), np.argsort(p).astype(np.int32)

def _lfa(lp, ext, nbr, dist2, f_in, d):
    h = d // 2
    p1 = lp["mlp1"]
    p2 = lp["mlp2"]
    psc = lp["shortcut"]
    scs = psc["gamma"] / jnp.sqrt(1.0 + _EPS)
    return _lfa_fused(
        ext, nbr, dist2, f_in,
        p1["W"].T, p1["b"].reshape(1, -1),
        _pool_args(lp["lse1"]["mlp"], lp["pool1"], h),
        _pool_args(lp["lse2"]["mlp"], lp["pool2"], h),
        p2["W"].T, psc["W"].T * scs[None, :],
        (p2["b"] + psc["b"] * scs + psc["beta"]).reshape(1, -1))


def kernel(input, params):
    b, n, _ = input.shape
    perm_np, inv_np = _perm_const(n)
    perm = jnp.asarray(perm_np)
    inv = jnp.asarray(inv_np)
    offs = (jnp.arange(b, dtype=jnp.int32) * n)[:, None]

    flat_in = input.reshape(b * n, 3)
    table_in = jnp.concatenate(
        [flat_in, jnp.zeros((b * n, 13), jnp.float32)], axis=1)
    pidx = (perm[None, :].astype(jnp.int32) + offs).reshape(-1)
    permuted = _sc_gather(table_in, pidx)
    coords = permuted[:, :3].reshape(b, n, 3)

    q2 = jnp.sum(coords * coords, axis=-1, keepdims=True)
    ones = jnp.ones_like(q2)
    qa = jnp.concatenate([coords, q2, ones], axis=-1)
    sa_t = jnp.concatenate([-2.0 * coords, ones, q2],
                           axis=-1).transpose(0, 2, 1)

    p0 = params["fc_start"]
    bn0 = params["bn_start"]
    sc0 = bn0["gamma"] / jnp.sqrt(1.0 + _EPS)
    x = _linear([permuted[:, :3]], [p0["W"].T * sc0[None, :]],
                p0["b"] * sc0 + bn0["beta"], "lrelu2")

    dims = [16, 64, 128, 256]
    levels = [n, n // 4, n // 16, n // 64]
    knns = [_knn(qa[:, :nl], sa_t[:, :, :nl], _K) for nl in levels]
    gidx_all = jnp.concatenate([
        (jnp.transpose(idx, (2, 0, 1))
         + (jnp.arange(b, dtype=jnp.int32) * n)[None, :, None]).reshape(-1)
        for idx, _ in knns])
    nbr_all = _sc_gather(permuted, gidx_all)
    nbrs, off = [], 0
    for nl in levels:
        nbrs.append(nbr_all[off:off + _K * b * nl].reshape(_K, b * nl, 16))
        off += _K * b * nl

    stack = []
    for li, lp in enumerate(params["encoder"]):
        nl = levels[li]
        ext = coords[:, :nl].reshape(b * nl, 3)
        dist2 = knns[li][1].reshape(b * nl, _K)
        x = _lfa(lp, ext, nbrs[li], dist2, x, dims[li])
        stack.append(x)
        c = x.shape[1]
        x = x.reshape(b, nl, c)[:, :nl // 4].reshape(b * nl // 4, c)

    pm = params["mid"]
    x = _linear([x], [pm["W"].T], pm["b"], "relu")

    dr = 256
    for dp in params["decoder"]:
        ns_ = n // dr
        nq_ = 4 * ns_
        nb1 = _knn(qa[:, :nq_], sa_t[:, :, :ns_], 1)
        gidx = (nb1[..., 0] + (jnp.arange(b, dtype=jnp.int32) * ns_)[:, None]
                ).reshape(-1)
        x_nb = _sc_gather(x, gidx)
        skip = stack.pop()
        scd = dp["gamma"] / jnp.sqrt(1.0 + _EPS)
        wt = dp["W"].T * scd[None, :]
        cnb = x.shape[1]
        x = _linear([x_nb, skip], [wt[:cnb], wt[cnb:]],
                    dp["b"] * scd + dp["beta"], "relu")
        dr //= 4

    tbl = jnp.concatenate([x, jnp.zeros((b * n, 8), jnp.float32)], axis=1)
    iidx = (inv[None, :].astype(jnp.int32) + offs).reshape(-1)
    x = _sc_gather(tbl, iidx)[:, :8]
    fe = params["fc_end"]
    sc1 = fe[0]["gamma"] / jnp.sqrt(1.0 + _EPS)
    sc2 = fe[1]["gamma"] / jnp.sqrt(1.0 + _EPS)
    x = _fc_end(x,
                fe[0]["W"].T * sc1[None, :],
                (fe[0]["b"] * sc1 + fe[0]["beta"]).reshape(1, -1),
                fe[1]["W"].T * sc2[None, :],
                (fe[1]["b"] * sc2 + fe[1]["beta"]).reshape(1, -1),
                fe[2]["W"].T, fe[2]["b"].reshape(1, -1))
    return x.reshape(b, n, 13).transpose(0, 2, 1)


# speedup vs baseline: 1.0629x; 1.0629x over previous
import functools

import jax
import jax.numpy as jnp
import numpy as np
from jax import lax
from jax.experimental import pallas as pl
from jax.experimental.pallas import tpu as pltpu
from jax.experimental.pallas import tpu_sc as plsc

_K = 16
_EPS = 1e-6
_HI = lax.Precision.HIGHEST
_NC, _NS = 2, 16
_NW = _NC * _NS


def _sc_gather(table, idx):
    m0 = idx.shape[0]
    d = table.shape[1]
    mpad = -(-m0 // (16 * _NW)) * (16 * _NW)
    if mpad != m0:
        idx = jnp.concatenate([idx, jnp.zeros((mpad - m0,), jnp.int32)])
    b_per_w = mpad // _NW
    ch = b_per_w
    while ch * (d + 1) > 65536:
        ch //= 2
    nchunk = b_per_w // ch
    mesh = plsc.VectorSubcoreMesh(core_axis_name="c", subcore_axis_name="s",
                                  num_cores=_NC, num_subcores=_NS)

    @functools.partial(
        pl.kernel,
        out_type=jax.ShapeDtypeStruct((mpad, d), jnp.float32),
        mesh=mesh,
        compiler_params=pltpu.CompilerParams(use_tc_tiling_on_sc=False),
        scratch_types=[
            pltpu.VMEM((ch,), jnp.int32),
            pltpu.VMEM((ch, d), jnp.float32),
            pltpu.SemaphoreType.DMA,
        ],
    )
    def gath(table_hbm, idx_hbm, out_hbm, idx_v, rows_v, sem):
        wid = lax.axis_index("s") * _NC + lax.axis_index("c")
        base = wid * b_per_w
        for c in range(nchunk):
            pltpu.sync_copy(idx_hbm.at[pl.ds(base + c * ch, ch)], idx_v)
            pltpu.async_copy(table_hbm.at[idx_v], rows_v, sem).wait()
            pltpu.sync_copy(rows_v, out_hbm.at[pl.ds(base + c * ch, ch)])

    out = gath(table, idx)
    return out[:m0] if mpad != m0 else out


def _linear(xs, ws, bias, act):
    m = xs[0].shape[0]
    co = ws[0].shape[1]
    tm = min(m, 512)
    nin = len(xs)
    b2 = bias.reshape(1, co)

    def body(*refs):
        o_ref = refs[-1]
        tot = None
        for i in range(nin):
            t = jnp.dot(refs[i][...], refs[nin + i][...],
                        preferred_element_type=jnp.float32, precision=_HI)
            tot = t if tot is None else tot + t
        tot = tot + refs[2 * nin][...]
        if act == "relu":
            tot = jnp.maximum(tot, 0.0)
        elif act == "lrelu2":
            tot = jnp.where(tot >= 0, tot, 0.2 * tot)
        elif act == "lrelu01":
            tot = jnp.where(tot >= 0, tot, 0.01 * tot)
        o_ref[...] = tot

    in_specs = ([pl.BlockSpec((tm, x.shape[1]), lambda i: (i, 0)) for x in xs]
                + [pl.BlockSpec(w.shape, lambda i: (0, 0)) for w in ws]
                + [pl.BlockSpec((1, co), lambda i: (0, 0))])
    return pl.pallas_call(
        body, grid=(m // tm,),
        in_specs=in_specs,
        out_specs=pl.BlockSpec((tm, co), lambda i: (i, 0)),
        out_shape=jax.ShapeDtypeStruct((m, co), jnp.float32),
    )(*xs, *ws, b2)


def _knn(qa, sa_t, k):
    b, nq, _ = qa.shape
    ns = sa_t.shape[2]
    tq = min(nq, 128)
    grid = (b, nq // tq)
    big = float("inf")

    if k == 1:
        def body1(qa_ref, sa_ref, idx_ref):
            d2 = jnp.dot(qa_ref[0], sa_ref[0],
                         preferred_element_type=jnp.float32, precision=_HI)
            iota = lax.broadcasted_iota(jnp.int32, (tq, ns), 1)
            m = jnp.min(d2, axis=1, keepdims=True)
            idx_ref[0] = jnp.min(jnp.where(d2 == m, iota, ns), axis=1,
                                 keepdims=True)

        return pl.pallas_call(
            body1, grid=grid,
            in_specs=[pl.BlockSpec((1, tq, 5), lambda bb, i: (bb, i, 0)),
                      pl.BlockSpec((1, 5, ns), lambda bb, i: (bb, 0, 0))],
            out_specs=pl.BlockSpec((1, tq, 1), lambda bb, i: (bb, i, 0)),
            out_shape=jax.ShapeDtypeStruct((b, nq, 1), jnp.int32),
        )(qa, sa_t)

    def body(qa_ref, sa_ref, idx_ref, dist_ref):
        d2 = jnp.dot(qa_ref[0], sa_ref[0],
                     preferred_element_type=jnp.float32, precision=_HI)
        ns4 = ns // 4
        iota = lax.broadcasted_iota(jnp.int32, (tq, ns4), 1)
        vs = [d2[:, c * ns4:(c + 1) * ns4] for c in range(4)]
        ids = [iota + c * ns4 for c in range(4)]

        def ce(i, j):
            sel = vs[i] <= vs[j]
            vs[i], vs[j] = (jnp.where(sel, vs[i], vs[j]),
                            jnp.where(sel, vs[j], vs[i]))
            ids[i], ids[j] = (jnp.where(sel, ids[i], ids[j]),
                              jnp.where(sel, ids[j], ids[i]))

        for i, j in ((0, 1), (2, 3), (0, 2), (1, 3), (1, 2)):
            ce(i, j)
        s0, s1, s2, s3 = vs
        i0, i1, i2 = ids[0], ids[1], ids[2]
        for t in range(k):
            m = jnp.min(s0, axis=1, keepdims=True)
            eq = s0 == m
            ii = jnp.min(jnp.where(eq, i0, ns), axis=1, keepdims=True)
            idx_ref[0, :, t:t + 1] = ii
            dist_ref[0, :, t:t + 1] = jnp.where(
                m > 0, jnp.sqrt(jnp.maximum(m, 0.0)), 0.0)
            if t < k - 1:
                s0 = jnp.where(eq, s1, s0)
                i0 = jnp.where(eq, i1, i0)
                s1 = jnp.where(eq, s2, s1)
                i1 = jnp.where(eq, i2, i1)
                s2 = jnp.where(eq, s3, s2)
                i2 = jnp.where(eq, ids[3], i2)
                s3 = jnp.where(eq, big, s3)

    return pl.pallas_call(
        body, grid=grid,
        in_specs=[pl.BlockSpec((1, tq, 5), lambda bb, i: (bb, i, 0)),
                  pl.BlockSpec((1, 5, ns), lambda bb, i: (bb, 0, 0))],
        out_specs=[pl.BlockSpec((1, tq, k), lambda bb, i: (bb, i, 0)),
                   pl.BlockSpec((1, tq, k), lambda bb, i: (bb, i, 0))],
        out_shape=[jax.ShapeDtypeStruct((b, nq, k), jnp.int32),
                   jax.ShapeDtypeStruct((b, nq, k), jnp.float32)],
    )(qa, sa_t)


def _lfa_fused(ext, nbr, dist, f_in, w1t, b1, args1, args2, w2t, wst, btot):
    m, d_in = f_in.shape
    co = w2t.shape[1]
    tp = min(m, 256)
    flat = [w1t, b1, *args1, *args2, w2t, wst, btot]

    def body(*refs):
        ext_ref, nbr_ref, dist_ref, f_ref = refs[:4]
        wrefs = refs[4:-1]
        o_ref = refs[-1]
        w1t_r, b1_r = wrefs[0:2]
        pa1 = wrefs[2:13]
        pa2 = wrefs[13:24]
        w2t_r, wst_r, btot_r = wrefs[24:27]
        ex = ext_ref[...]
        f = f_ref[...]
        x1 = jnp.dot(f, w1t_r[...], preferred_element_type=jnp.float32,
                     precision=_HI) + b1_r[...]
        x1 = jnp.where(x1 >= 0, x1, 0.2 * x1)

        def pool(ft, pa):
            (a_r, bm_r, wd_r, esc_r, esh_r, w1_r, w2_r, wml_r, wmr_r,
             psc_r, psh_r) = pa
            ext_a = jnp.dot(ex, a_r[...], preferred_element_type=jnp.float32,
                            precision=_HI)
            f_w2 = jnp.dot(ft, w2_r[...], preferred_element_type=jnp.float32,
                           precision=_HI)
            encs, logs = [], []
            for kk in range(_K):
                nb = nbr_ref[kk, :, 0:3]
                dk = dist_ref[:, kk:kk + 1]
                enc = ext_a + jnp.dot(nb, bm_r[...],
                                      preferred_element_type=jnp.float32,
                                      precision=_HI) + dk * wd_r[...]
                enc = jnp.maximum(enc * esc_r[...] + esh_r[...], 0.0)
                encs.append(enc)
                logs.append(jnp.dot(enc, w1_r[...],
                                    preferred_element_type=jnp.float32,
                                    precision=_HI) + f_w2)
            mx = logs[0]
            for kk in range(1, _K):
                mx = jnp.maximum(mx, logs[kk])
            den = None
            accl = None
            for kk in range(_K):
                e = jnp.exp(logs[kk] - mx)
                den = e if den is None else den + e
                t = e * encs[kk]
                accl = t if accl is None else accl + t
            out = (jnp.dot(accl / den, wml_r[...],
                           preferred_element_type=jnp.float32, precision=_HI)
                   + jnp.dot(ft, wmr_r[...],
                             preferred_element_type=jnp.float32,
                             precision=_HI))
            return jnp.maximum(out * psc_r[...] + psh_r[...], 0.0)

        z1 = pool(x1, pa1)
        z2 = pool(z1, pa2)
        out = (jnp.dot(z2, w2t_r[...], preferred_element_type=jnp.float32,
                       precision=_HI)
               + jnp.dot(f, wst_r[...], preferred_element_type=jnp.float32,
                         precision=_HI) + btot_r[...])
        o_ref[...] = jnp.where(out >= 0, out, 0.01 * out)

    full = lambda arr: pl.BlockSpec(arr.shape, lambda i: tuple(0 for _ in arr.shape))
    in_specs = [pl.BlockSpec((tp, 3), lambda i: (i, 0)),
                pl.BlockSpec((_K, tp, 16), lambda i: (0, i, 0)),
                pl.BlockSpec((tp, _K), lambda i: (i, 0)),
                pl.BlockSpec((tp, d_in), lambda i: (i, 0))] + [full(w) for w in flat]
    return pl.pallas_call(
        body, grid=(m // tp,),
        in_specs=in_specs,
        out_specs=pl.BlockSpec((tp, co), lambda i: (i, 0)),
        out_shape=jax.ShapeDtypeStruct((m, co), jnp.float32),
    )(ext, nbr, dist, f_in, *flat)


def _pool_args(lse_mlp, pool, h):
    w = lse_mlp["W"]
    a = (w[:, 0:3] + w[:, 6:9]).T
    bm = (w[:, 3:6] - w[:, 6:9]).T
    wd = w[:, 9][None, :]
    esc = (lse_mlp["gamma"] / jnp.sqrt(1.0 + _EPS))[None, :]
    esh = (lse_mlp["b"] * esc[0] + lse_mlp["beta"])[None, :]
    wst = pool["score_W"].T
    w1 = wst[:h, :h]
    w2 = wst[h:, :h]
    pm = pool["mlp"]
    wmt = pm["W"].T
    psc = (pm["gamma"] / jnp.sqrt(1.0 + _EPS))[None, :]
    psh = (pm["b"] * psc[0] + pm["beta"])[None, :]
    return (a, bm, wd, esc, esh, w1, w2, wmt[:h], wmt[h:], psc, psh)


def _fc_end(x, w1, b1, w2, b2, w3, b3):
    m = x.shape[0]
    tm = min(m, 1024)

    def body(x_ref, w1_ref, b1_ref, w2_ref, b2_ref, w3_ref, b3_ref, o_ref):
        t = jnp.maximum(jnp.dot(x_ref[...], w1_ref[...],
                                preferred_element_type=jnp.float32,
                                precision=_HI) + b1_ref[...], 0.0)
        t = jnp.maximum(jnp.dot(t, w2_ref[...],
                                preferred_element_type=jnp.float32,
                                precision=_HI) + b2_ref[...], 0.0)
        o_ref[...] = jnp.dot(t, w3_ref[...],
                             preferred_element_type=jnp.float32,
                             precision=_HI) + b3_ref[...]

    full = lambda arr: pl.BlockSpec(arr.shape, lambda i: tuple(0 for _ in arr.shape))
    return pl.pallas_call(
        body, grid=(m // tm,),
        in_specs=[pl.BlockSpec((tm, x.shape[1]), lambda i: (i, 0)),
                  full(w1), full(b1), full(w2), full(b2), full(w3), full(b3)],
        out_specs=pl.BlockSpec((tm, 13), lambda i: (i, 0)),
        out_shape=jax.ShapeDtypeStruct((m, 13), jnp.float32),
    )(x, w1, b1, w2, b2, w3, b3)


@functools.lru_cache(maxsize=None)
def _perm_const(n):
    with jax.ensure_compile_time_eval():
        p = np.asarray(jax.random.permutation(jax.random.key(1), n))
    return p.astype(np.int32), np.argsort(p).astype(np.int32)

def _lfa(lp, ext, nbr, dist2, f_in, d):
    h = d // 2
    p1 = lp["mlp1"]
    p2 = lp["mlp2"]
    psc = lp["shortcut"]
    scs = psc["gamma"] / jnp.sqrt(1.0 + _EPS)
    return _lfa_fused(
        ext, nbr, dist2, f_in,
        p1["W"].T, p1["b"].reshape(1, -1),
        _pool_args(lp["lse1"]["mlp"], lp["pool1"], h),
        _pool_args(lp["lse2"]["mlp"], lp["pool2"], h),
        p2["W"].T, psc["W"].T * scs[None, :],
        (p2["b"] + psc["b"] * scs + psc["beta"]).reshape(1, -1))


def kernel(input, params):
    b, n, _ = input.shape
    perm_np, inv_np = _perm_const(n)
    perm = jnp.asarray(perm_np)
    inv = jnp.asarray(inv_np)
    offs = (jnp.arange(b, dtype=jnp.int32) * n)[:, None]

    flat_in = input.reshape(b * n, 3)
    table_in = jnp.concatenate(
        [flat_in, jnp.zeros((b * n, 13), jnp.float32)], axis=1)
    pidx = (perm[None, :].astype(jnp.int32) + offs).reshape(-1)
    permuted = _sc_gather(table_in, pidx)
    coords = permuted[:, :3].reshape(b, n, 3)

    q2 = jnp.sum(coords * coords, axis=-1, keepdims=True)
    ones = jnp.ones_like(q2)
    qa = jnp.concatenate([coords, q2, ones], axis=-1)
    sa_t = jnp.concatenate([-2.0 * coords, ones, q2],
                           axis=-1).transpose(0, 2, 1)

    p0 = params["fc_start"]
    bn0 = params["bn_start"]
    sc0 = bn0["gamma"] / jnp.sqrt(1.0 + _EPS)
    x = _linear([permuted[:, :3]], [p0["W"].T * sc0[None, :]],
                p0["b"] * sc0 + bn0["beta"], "lrelu2")

    dims = [16, 64, 128, 256]
    levels = [n, n // 4, n // 16, n // 64]
    knns = [_knn(qa[:, :nl], sa_t[:, :, :nl], _K) for nl in levels]
    gidx_all = jnp.concatenate([
        (jnp.transpose(idx, (2, 0, 1))
         + (jnp.arange(b, dtype=jnp.int32) * n)[None, :, None]).reshape(-1)
        for idx, _ in knns])
    nbr_all = _sc_gather(permuted, gidx_all)
    nbrs, off = [], 0
    for nl in levels:
        nbrs.append(nbr_all[off:off + _K * b * nl].reshape(_K, b * nl, 16))
        off += _K * b * nl

    stack = []
    for li, lp in enumerate(params["encoder"]):
        nl = levels[li]
        ext = coords[:, :nl].reshape(b * nl, 3)
        dist2 = knns[li][1].reshape(b * nl, _K)
        x = _lfa(lp, ext, nbrs[li], dist2, x, dims[li])
        stack.append(x)
        c = x.shape[1]
        x = x.reshape(b, nl, c)[:, :nl // 4].reshape(b * nl // 4, c)

    pm = params["mid"]
    x = _linear([x], [pm["W"].T], pm["b"], "relu")

    dr = 256
    for dp in params["decoder"]:
        ns_ = n // dr
        nq_ = 4 * ns_
        nb1 = _knn(qa[:, :nq_], sa_t[:, :, :ns_], 1)
        gidx = (nb1[..., 0] + (jnp.arange(b, dtype=jnp.int32) * ns_)[:, None]
                ).reshape(-1)
        x_nb = _sc_gather(x, gidx)
        skip = stack.pop()
        scd = dp["gamma"] / jnp.sqrt(1.0 + _EPS)
        wt = dp["W"].T * scd[None, :]
        cnb = x.shape[1]
        x = _linear([x_nb, skip], [wt[:cnb], wt[cnb:]],
                    dp["b"] * scd + dp["beta"], "relu")
        dr //= 4

    tbl = jnp.concatenate([x, jnp.zeros((b * n, 8), jnp.float32)], axis=1)
    iidx = (inv[None, :].astype(jnp.int32) + offs).reshape(-1)
    x = _sc_gather(tbl, iidx)[:, :8]
    fe = params["fc_end"]
    sc1 = fe[0]["gamma"] / jnp.sqrt(1.0 + _EPS)
    sc2 = fe[1]["gamma"] / jnp.sqrt(1.0 + _EPS)
    x = _fc_end(x,
                fe[0]["W"].T * sc1[None, :],
                (fe[0]["b"] * sc1 + fe[0]["beta"]).reshape(1, -1),
                fe[1]["W"].T * sc2[None, :],
                (fe[1]["b"] * sc2 + fe[1]["beta"]).reshape(1, -1),
                fe[2]["W"].T, fe[2]["b"].reshape(1, -1))
    return x.reshape(b, n, 13).transpose(0, 2, 1)


# speedup vs baseline: 1.0921x; 1.0275x over previous
import functools

import jax
import jax.numpy as jnp
import numpy as np
from jax import lax
from jax.experimental import pallas as pl
from jax.experimental.pallas import tpu as pltpu
from jax.experimental.pallas import tpu_sc as plsc

_K = 16
_EPS = 1e-6
_HI = lax.Precision.HIGHEST
_NC, _NS = 2, 16
_NW = _NC * _NS


def _sc_gather(table, idx):
    m0 = idx.shape[0]
    d = table.shape[1]
    mpad = -(-m0 // (16 * _NW)) * (16 * _NW)
    if mpad != m0:
        idx = jnp.concatenate([idx, jnp.zeros((mpad - m0,), jnp.int32)])
    b_per_w = mpad // _NW
    ch = b_per_w
    while ch * (d + 1) > 65536:
        ch //= 2
    nchunk = b_per_w // ch
    mesh = plsc.VectorSubcoreMesh(core_axis_name="c", subcore_axis_name="s",
                                  num_cores=_NC, num_subcores=_NS)

    @functools.partial(
        pl.kernel,
        out_type=jax.ShapeDtypeStruct((mpad, d), jnp.float32),
        mesh=mesh,
        compiler_params=pltpu.CompilerParams(use_tc_tiling_on_sc=False),
        scratch_types=[
            pltpu.VMEM((ch,), jnp.int32),
            pltpu.VMEM((ch, d), jnp.float32),
            pltpu.SemaphoreType.DMA,
        ],
    )
    def gath(table_hbm, idx_hbm, out_hbm, idx_v, rows_v, sem):
        wid = lax.axis_index("s") * _NC + lax.axis_index("c")
        base = wid * b_per_w
        for c in range(nchunk):
            pltpu.sync_copy(idx_hbm.at[pl.ds(base + c * ch, ch)], idx_v)
            pltpu.async_copy(table_hbm.at[idx_v], rows_v, sem).wait()
            pltpu.sync_copy(rows_v, out_hbm.at[pl.ds(base + c * ch, ch)])

    out = gath(table, idx)
    return out[:m0] if mpad != m0 else out


def _linear(xs, ws, bias, act):
    m = xs[0].shape[0]
    co = ws[0].shape[1]
    tm = min(m, 512)
    nin = len(xs)
    b2 = bias.reshape(1, co)

    def body(*refs):
        o_ref = refs[-1]
        tot = None
        for i in range(nin):
            t = jnp.dot(refs[i][...], refs[nin + i][...],
                        preferred_element_type=jnp.float32, precision=_HI)
            tot = t if tot is None else tot + t
        tot = tot + refs[2 * nin][...]
        if act == "relu":
            tot = jnp.maximum(tot, 0.0)
        elif act == "lrelu2":
            tot = jnp.where(tot >= 0, tot, 0.2 * tot)
        elif act == "lrelu01":
            tot = jnp.where(tot >= 0, tot, 0.01 * tot)
        o_ref[...] = tot

    in_specs = ([pl.BlockSpec((tm, x.shape[1]), lambda i: (i, 0)) for x in xs]
                + [pl.BlockSpec(w.shape, lambda i: (0, 0)) for w in ws]
                + [pl.BlockSpec((1, co), lambda i: (0, 0))])
    return pl.pallas_call(
        body, grid=(m // tm,),
        in_specs=in_specs,
        out_specs=pl.BlockSpec((tm, co), lambda i: (i, 0)),
        out_shape=jax.ShapeDtypeStruct((m, co), jnp.float32),
    )(*xs, *ws, b2)


def _knn(qa, sa_t, k):
    b, nq, _ = qa.shape
    ns = sa_t.shape[2]
    tq = min(nq, 256)
    grid = (b, nq // tq)
    big = float("inf")

    if k == 1:
        def body1(qa_ref, sa_ref, idx_ref):
            d2 = jnp.dot(qa_ref[0], sa_ref[0],
                         preferred_element_type=jnp.float32, precision=_HI)
            iota = lax.broadcasted_iota(jnp.int32, (tq, ns), 1)
            m = jnp.min(d2, axis=1, keepdims=True)
            idx_ref[0] = jnp.min(jnp.where(d2 == m, iota, ns), axis=1,
                                 keepdims=True)

        return pl.pallas_call(
            body1, grid=grid,
            in_specs=[pl.BlockSpec((1, tq, 5), lambda bb, i: (bb, i, 0)),
                      pl.BlockSpec((1, 5, ns), lambda bb, i: (bb, 0, 0))],
            out_specs=pl.BlockSpec((1, tq, 1), lambda bb, i: (bb, i, 0)),
            out_shape=jax.ShapeDtypeStruct((b, nq, 1), jnp.int32),
        )(qa, sa_t)

    def body(qa_ref, sa_ref, idx_ref, dist_ref):
        d2 = jnp.dot(qa_ref[0], sa_ref[0],
                     preferred_element_type=jnp.float32, precision=_HI)
        ns4 = ns // 4
        iota = lax.broadcasted_iota(jnp.int32, (tq, ns4), 1)
        vs = [d2[:, c * ns4:(c + 1) * ns4] for c in range(4)]
        ids = [iota + c * ns4 for c in range(4)]

        def ce(i, j):
            sel = vs[i] <= vs[j]
            vs[i], vs[j] = (jnp.where(sel, vs[i], vs[j]),
                            jnp.where(sel, vs[j], vs[i]))
            ids[i], ids[j] = (jnp.where(sel, ids[i], ids[j]),
                              jnp.where(sel, ids[j], ids[i]))

        for i, j in ((0, 1), (2, 3), (0, 2), (1, 3), (1, 2)):
            ce(i, j)
        s0, s1, s2, s3 = vs
        i0, i1, i2 = ids[0], ids[1], ids[2]
        for t in range(k):
            m = jnp.min(s0, axis=1, keepdims=True)
            eq = s0 == m
            ii = jnp.min(jnp.where(eq, i0, ns), axis=1, keepdims=True)
            idx_ref[0, :, t:t + 1] = ii
            dist_ref[0, :, t:t + 1] = jnp.where(
                m > 0, jnp.sqrt(jnp.maximum(m, 0.0)), 0.0)
            if t < k - 1:
                s0 = jnp.where(eq, s1, s0)
                i0 = jnp.where(eq, i1, i0)
                s1 = jnp.where(eq, s2, s1)
                i1 = jnp.where(eq, i2, i1)
                s2 = jnp.where(eq, s3, s2)
                i2 = jnp.where(eq, ids[3], i2)
                s3 = jnp.where(eq, big, s3)

    return pl.pallas_call(
        body, grid=grid,
        in_specs=[pl.BlockSpec((1, tq, 5), lambda bb, i: (bb, i, 0)),
                  pl.BlockSpec((1, 5, ns), lambda bb, i: (bb, 0, 0))],
        out_specs=[pl.BlockSpec((1, tq, k), lambda bb, i: (bb, i, 0)),
                   pl.BlockSpec((1, tq, k), lambda bb, i: (bb, i, 0))],
        out_shape=[jax.ShapeDtypeStruct((b, nq, k), jnp.int32),
                   jax.ShapeDtypeStruct((b, nq, k), jnp.float32)],
    )(qa, sa_t)


def _lfa_fused(ext, nbr, dist, f_in, w1t, b1, args1, args2, w2t, wst, btot):
    m, d_in = f_in.shape
    co = w2t.shape[1]
    tp = min(m, 256)
    flat = [w1t, b1, *args1, *args2, w2t, wst, btot]

    def body(*refs):
        ext_ref, nbr_ref, dist_ref, f_ref = refs[:4]
        wrefs = refs[4:-1]
        o_ref = refs[-1]
        w1t_r, b1_r = wrefs[0:2]
        pa1 = wrefs[2:13]
        pa2 = wrefs[13:24]
        w2t_r, wst_r, btot_r = wrefs[24:27]
        ex = ext_ref[...]
        f = f_ref[...]
        x1 = jnp.dot(f, w1t_r[...], preferred_element_type=jnp.float32,
                     precision=_HI) + b1_r[...]
        x1 = jnp.where(x1 >= 0, x1, 0.2 * x1)

        def pool(ft, pa):
            (a_r, bm_r, wd_r, esc_r, esh_r, w1_r, w2_r, wml_r, wmr_r,
             psc_r, psh_r) = pa
            ext_a = jnp.dot(ex, a_r[...], preferred_element_type=jnp.float32,
                            precision=_HI)
            f_w2 = jnp.dot(ft, w2_r[...], preferred_element_type=jnp.float32,
                           precision=_HI)
            encs, logs = [], []
            for kk in range(_K):
                nb = nbr_ref[kk, :, 0:3]
                dk = dist_ref[:, kk:kk + 1]
                enc = ext_a + jnp.dot(nb, bm_r[...],
                                      preferred_element_type=jnp.float32,
                                      precision=_HI) + dk * wd_r[...]
                enc = jnp.maximum(enc * esc_r[...] + esh_r[...], 0.0)
                encs.append(enc)
                logs.append(jnp.dot(enc, w1_r[...],
                                    preferred_element_type=jnp.float32,
                                    precision=_HI) + f_w2)
            mx = logs[0]
            for kk in range(1, _K):
                mx = jnp.maximum(mx, logs[kk])
            den = None
            accl = None
            for kk in range(_K):
                e = jnp.exp(logs[kk] - mx)
                den = e if den is None else den + e
                t = e * encs[kk]
                accl = t if accl is None else accl + t
            out = (jnp.dot(accl / den, wml_r[...],
                           preferred_element_type=jnp.float32, precision=_HI)
                   + jnp.dot(ft, wmr_r[...],
                             preferred_element_type=jnp.float32,
                             precision=_HI))
            return jnp.maximum(out * psc_r[...] + psh_r[...], 0.0)

        z1 = pool(x1, pa1)
        z2 = pool(z1, pa2)
        out = (jnp.dot(z2, w2t_r[...], preferred_element_type=jnp.float32,
                       precision=_HI)
               + jnp.dot(f, wst_r[...], preferred_element_type=jnp.float32,
                         precision=_HI) + btot_r[...])
        o_ref[...] = jnp.where(out >= 0, out, 0.01 * out)

    full = lambda arr: pl.BlockSpec(arr.shape, lambda i: tuple(0 for _ in arr.shape))
    in_specs = [pl.BlockSpec((tp, 3), lambda i: (i, 0)),
                pl.BlockSpec((_K, tp, 16), lambda i: (0, i, 0)),
                pl.BlockSpec((tp, _K), lambda i: (i, 0)),
                pl.BlockSpec((tp, d_in), lambda i: (i, 0))] + [full(w) for w in flat]
    return pl.pallas_call(
        body, grid=(m // tp,),
        in_specs=in_specs,
        out_specs=pl.BlockSpec((tp, co), lambda i: (i, 0)),
        out_shape=jax.ShapeDtypeStruct((m, co), jnp.float32),
    )(ext, nbr, dist, f_in, *flat)


def _pool_args(lse_mlp, pool, h):
    w = lse_mlp["W"]
    a = (w[:, 0:3] + w[:, 6:9]).T
    bm = (w[:, 3:6] - w[:, 6:9]).T
    wd = w[:, 9][None, :]
    esc = (lse_mlp["gamma"] / jnp.sqrt(1.0 + _EPS))[None, :]
    esh = (lse_mlp["b"] * esc[0] + lse_mlp["beta"])[None, :]
    wst = pool["score_W"].T
    w1 = wst[:h, :h]
    w2 = wst[h:, :h]
    pm = pool["mlp"]
    wmt = pm["W"].T
    psc = (pm["gamma"] / jnp.sqrt(1.0 + _EPS))[None, :]
    psh = (pm["b"] * psc[0] + pm["beta"])[None, :]
    return (a, bm, wd, esc, esh, w1, w2, wmt[:h], wmt[h:], psc, psh)


def _fc_end(x, w1, b1, w2, b2, w3, b3):
    m = x.shape[0]
    tm = min(m, 1024)

    def body(x_ref, w1_ref, b1_ref, w2_ref, b2_ref, w3_ref, b3_ref, o_ref):
        t = jnp.maximum(jnp.dot(x_ref[...], w1_ref[...],
                                preferred_element_type=jnp.float32,
                                precision=_HI) + b1_ref[...], 0.0)
        t = jnp.maximum(jnp.dot(t, w2_ref[...],
                                preferred_element_type=jnp.float32,
                                precision=_HI) + b2_ref[...], 0.0)
        o_ref[...] = jnp.dot(t, w3_ref[...],
                             preferred_element_type=jnp.float32,
                             precision=_HI) + b3_ref[...]

    full = lambda arr: pl.BlockSpec(arr.shape, lambda i: tuple(0 for _ in arr.shape))
    return pl.pallas_call(
        body, grid=(m // tm,),
        in_specs=[pl.BlockSpec((tm, x.shape[1]), lambda i: (i, 0)),
                  full(w1), full(b1), full(w2), full(b2), full(w3), full(b3)],
        out_specs=pl.BlockSpec((tm, 13), lambda i: (i, 0)),
        out_shape=jax.ShapeDtypeStruct((m, 13), jnp.float32),
    )(x, w1, b1, w2, b2, w3, b3)


@functools.lru_cache(maxsize=None)
def _perm_const(n):
    with jax.ensure_compile_time_eval():
        p = np.asarray(jax.random.permutation(jax.random.key(1), n))
    return p.astype(np.int32), np.argsort(p).astype(np.int32)

def _lfa(lp, ext, nbr, dist2, f_in, d):
    h = d // 2
    p1 = lp["mlp1"]
    p2 = lp["mlp2"]
    psc = lp["shortcut"]
    scs = psc["gamma"] / jnp.sqrt(1.0 + _EPS)
    return _lfa_fused(
        ext, nbr, dist2, f_in,
        p1["W"].T, p1["b"].reshape(1, -1),
        _pool_args(lp["lse1"]["mlp"], lp["pool1"], h),
        _pool_args(lp["lse2"]["mlp"], lp["pool2"], h),
        p2["W"].T, psc["W"].T * scs[None, :],
        (p2["b"] + psc["b"] * scs + psc["beta"]).reshape(1, -1))


def kernel(input, params):
    b, n, _ = input.shape
    perm_np, inv_np = _perm_const(n)
    perm = jnp.asarray(perm_np)
    inv = jnp.asarray(inv_np)
    offs = (jnp.arange(b, dtype=jnp.int32) * n)[:, None]

    flat_in = input.reshape(b * n, 3)
    table_in = jnp.concatenate(
        [flat_in, jnp.zeros((b * n, 13), jnp.float32)], axis=1)
    pidx = (perm[None, :].astype(jnp.int32) + offs).reshape(-1)
    permuted = _sc_gather(table_in, pidx)
    coords = permuted[:, :3].reshape(b, n, 3)

    q2 = jnp.sum(coords * coords, axis=-1, keepdims=True)
    ones = jnp.ones_like(q2)
    qa = jnp.concatenate([coords, q2, ones], axis=-1)
    sa_t = jnp.concatenate([-2.0 * coords, ones, q2],
                           axis=-1).transpose(0, 2, 1)

    p0 = params["fc_start"]
    bn0 = params["bn_start"]
    sc0 = bn0["gamma"] / jnp.sqrt(1.0 + _EPS)
    x = _linear([permuted[:, :3]], [p0["W"].T * sc0[None, :]],
                p0["b"] * sc0 + bn0["beta"], "lrelu2")

    dims = [16, 64, 128, 256]
    levels = [n, n // 4, n // 16, n // 64]
    knns = [_knn(qa[:, :nl], sa_t[:, :, :nl], _K) for nl in levels]
    gidx_all = jnp.concatenate([
        (jnp.transpose(idx, (2, 0, 1))
         + (jnp.arange(b, dtype=jnp.int32) * n)[None, :, None]).reshape(-1)
        for idx, _ in knns])
    nbr_all = _sc_gather(permuted, gidx_all)
    nbrs, off = [], 0
    for nl in levels:
        nbrs.append(nbr_all[off:off + _K * b * nl].reshape(_K, b * nl, 16))
        off += _K * b * nl

    stack = []
    for li, lp in enumerate(params["encoder"]):
        nl = levels[li]
        ext = coords[:, :nl].reshape(b * nl, 3)
        dist2 = knns[li][1].reshape(b * nl, _K)
        x = _lfa(lp, ext, nbrs[li], dist2, x, dims[li])
        stack.append(x)
        c = x.shape[1]
        x = x.reshape(b, nl, c)[:, :nl // 4].reshape(b * nl // 4, c)

    pm = params["mid"]
    x = _linear([x], [pm["W"].T], pm["b"], "relu")

    dr = 256
    for dp in params["decoder"]:
        ns_ = n // dr
        nq_ = 4 * ns_
        nb1 = _knn(qa[:, :nq_], sa_t[:, :, :ns_], 1)
        gidx = (nb1[..., 0] + (jnp.arange(b, dtype=jnp.int32) * ns_)[:, None]
                ).reshape(-1)
        x_nb = _sc_gather(x, gidx)
        skip = stack.pop()
        scd = dp["gamma"] / jnp.sqrt(1.0 + _EPS)
        wt = dp["W"].T * scd[None, :]
        cnb = x.shape[1]
        x = _linear([x_nb, skip], [wt[:cnb], wt[cnb:]],
                    dp["b"] * scd + dp["beta"], "relu")
        dr //= 4

    tbl = jnp.concatenate([x, jnp.zeros((b * n, 8), jnp.float32)], axis=1)
    iidx = (inv[None, :].astype(jnp.int32) + offs).reshape(-1)
    x = _sc_gather(tbl, iidx)[:, :8]
    fe = params["fc_end"]
    sc1 = fe[0]["gamma"] / jnp.sqrt(1.0 + _EPS)
    sc2 = fe[1]["gamma"] / jnp.sqrt(1.0 + _EPS)
    x = _fc_end(x,
                fe[0]["W"].T * sc1[None, :],
                (fe[0]["b"] * sc1 + fe[0]["beta"]).reshape(1, -1),
                fe[1]["W"].T * sc2[None, :],
                (fe[1]["b"] * sc2 + fe[1]["beta"]).reshape(1, -1),
                fe[2]["W"].T, fe[2]["b"].reshape(1, -1))
    return x.reshape(b, n, 13).transpose(0, 2, 1)
